# baseline (speedup 1.0000x reference)
import jax
import jax.numpy as jnp
from jax.experimental import pallas as pl
from jax.experimental.pallas import tpu as pltpu

_BLOCK_ROWS = 2000


def _mlp_block(x_ref, w1_ref, b1_ref, w2_ref, b2_ref, o_ref):
    h = jnp.dot(x_ref[...], w1_ref[...], preferred_element_type=jnp.float32)
    h = jnp.maximum(h + b1_ref[...], 0.0)
    o_ref[...] = (
        jnp.dot(h, w2_ref[...], preferred_element_type=jnp.float32) + b2_ref[...]
    )


def kernel(x, W1, b1, W2, b2):
    n, d_in = x.shape
    d_hid = W1.shape[1]
    n_cls = W2.shape[1]
    b1 = b1.reshape(1, d_hid)
    b2 = b2.reshape(1, n_cls)
    grid = (n // _BLOCK_ROWS,)
    return pl.pallas_call(
        _mlp_block,
        grid=grid,
        in_specs=[
            pl.BlockSpec((_BLOCK_ROWS, d_in), lambda i: (i, 0)),
            pl.BlockSpec((d_in, d_hid), lambda i: (0, 0)),
            pl.BlockSpec((1, d_hid), lambda i: (0, 0)),
            pl.BlockSpec((d_hid, n_cls), lambda i: (0, 0)),
            pl.BlockSpec((1, n_cls), lambda i: (0, 0)),
        ],
        out_specs=pl.BlockSpec((_BLOCK_ROWS, n_cls), lambda i: (i, 0)),
        out_shape=jax.ShapeDtypeStruct((n, n_cls), jnp.float32),
        compiler_params=pltpu.CompilerParams(
            dimension_semantics=("parallel",),
        ),
    )(x, W1, b1, W2, b2)


# speedup vs baseline: 1.1564x; 1.1564x over previous
import jax
import jax.numpy as jnp
from jax.experimental import pallas as pl
from jax.experimental.pallas import tpu as pltpu

_BLOCK_ROWS = 4000


def _mlp_block(x_ref, w1_ref, b1_ref, w2t_ref, b2_ref, o_ref):
    h = jnp.dot(x_ref[...], w1_ref[...], preferred_element_type=jnp.float32)
    h = jnp.maximum(h + b1_ref[...], 0.0)
    n_cls = w2t_ref.shape[0]
    cols = [
        jnp.sum(h * w2t_ref[c : c + 1, :], axis=1, keepdims=True)
        for c in range(n_cls)
    ]
    o_ref[...] = jnp.concatenate(cols, axis=1) + b2_ref[...]


def kernel(x, W1, b1, W2, b2):
    n, d_in = x.shape
    d_hid = W1.shape[1]
    n_cls = W2.shape[1]
    b1 = b1.reshape(1, d_hid)
    b2 = b2.reshape(1, n_cls)
    W2t = W2.T
    grid = (n // _BLOCK_ROWS,)
    return pl.pallas_call(
        _mlp_block,
        grid=grid,
        in_specs=[
            pl.BlockSpec((_BLOCK_ROWS, d_in), lambda i: (i, 0)),
            pl.BlockSpec((d_in, d_hid), lambda i: (0, 0)),
            pl.BlockSpec((1, d_hid), lambda i: (0, 0)),
            pl.BlockSpec((n_cls, d_hid), lambda i: (0, 0)),
            pl.BlockSpec((1, n_cls), lambda i: (0, 0)),
        ],
        out_specs=pl.BlockSpec((_BLOCK_ROWS, n_cls), lambda i: (i, 0)),
        out_shape=jax.ShapeDtypeStruct((n, n_cls), jnp.float32),
        compiler_params=pltpu.CompilerParams(
            dimension_semantics=("parallel",),
        ),
    )(x, W1, b1, W2t, b2)
